# baseline (speedup 1.0000x reference)
import jax
import jax.numpy as jnp
from jax.experimental import pallas as pl

_NUM_RAYS = 1024
_NUM_LIGHTS = 32
_BATCH = 4096
_NB = 8
_ROWS = _NB * _NUM_RAYS


def _light_kernel(idx_ref, pt_ref, ld_ref, li_ref):
    idx_lane = idx_ref[0].astype(jnp.float32)
    row_b = jax.lax.broadcasted_iota(jnp.int32, (_ROWS, 1), 0) >> 10
    oh_b = (jax.lax.broadcasted_iota(jnp.int32, (_ROWS, _NB), 1) == row_b)
    lid = jnp.sum(oh_b.astype(jnp.float32) * idx_lane, axis=1, keepdims=True)
    lid = lid.astype(jnp.int32)
    oh = (jax.lax.broadcasted_iota(jnp.int32, (_ROWS, _NUM_LIGHTS), 1) == lid)
    oh = oh.astype(jnp.float32)
    pt = pt_ref[...]
    x = jnp.sum(oh * pt[0:1, :], axis=1, keepdims=True)
    y = jnp.sum(oh * pt[1:2, :], axis=1, keepdims=True)
    z = -jnp.abs(jnp.sum(oh * pt[2:3, :], axis=1, keepdims=True))
    inten = jnp.abs(jnp.sum(oh * pt[3:4, :], axis=1, keepdims=True))
    n = jnp.sqrt(x * x + y * y + z * z)
    inv = 1.0 / jnp.maximum(n, 1e-12)
    xn, yn, zn = x * inv, y * inv, z * inv
    c = jax.lax.broadcasted_iota(jnp.int32, (_ROWS, 3), 1)
    ld_ref[...] = jnp.where(c == 0, xn, jnp.where(c == 1, yn, zn))
    li_ref[...] = jnp.broadcast_to(inten, (_ROWS, 3))


def kernel(idx, light_direction_xy, light_direction_z, light_intensity):
    params_t = jnp.concatenate(
        [light_direction_xy, light_direction_z, light_intensity], axis=1
    ).T
    grid = _BATCH // _NB
    idx3 = idx.reshape(grid, 1, _NB)

    out_ld, out_li = pl.pallas_call(
        _light_kernel,
        grid=(grid,),
        in_specs=[
            pl.BlockSpec((1, 1, _NB), lambda i: (i, 0, 0)),
            pl.BlockSpec((4, _NUM_LIGHTS), lambda i: (0, 0)),
        ],
        out_specs=[
            pl.BlockSpec((_ROWS, 3), lambda i: (i, 0)),
            pl.BlockSpec((_ROWS, 3), lambda i: (i, 0)),
        ],
        out_shape=[
            jax.ShapeDtypeStruct((_BATCH * _NUM_RAYS, 3), jnp.float32),
            jax.ShapeDtypeStruct((_BATCH * _NUM_RAYS, 3), jnp.float32),
        ],
    )(idx3, params_t)
    return (out_ld, out_li)


# speedup vs baseline: 46.7589x; 46.7589x over previous
import jax
import jax.numpy as jnp
from jax.experimental import pallas as pl

_NUM_RAYS = 1024
_NUM_LIGHTS = 32
_BATCH = 4096
_NB = 512


def _light_kernel(idx_ref, pt_ref, ld_ref, li_ref):
    idxv = idx_ref[...]
    lanes = jax.lax.broadcasted_iota(jnp.int32, (_NB, _NUM_LIGHTS), 1)
    oh = (lanes == idxv).astype(jnp.float32)
    pt = pt_ref[...]
    x = jnp.sum(oh * pt[0:1, :], axis=1, keepdims=True)
    y = jnp.sum(oh * pt[1:2, :], axis=1, keepdims=True)
    z = -jnp.abs(jnp.sum(oh * pt[2:3, :], axis=1, keepdims=True))
    inten = jnp.abs(jnp.sum(oh * pt[3:4, :], axis=1, keepdims=True))
    n = jnp.sqrt(x * x + y * y + z * z)
    inv = 1.0 / jnp.maximum(n, 1e-12)
    shape = (_NB, _NUM_RAYS)
    ld_ref[0] = jnp.broadcast_to(x * inv, shape)
    ld_ref[1] = jnp.broadcast_to(y * inv, shape)
    ld_ref[2] = jnp.broadcast_to(z * inv, shape)
    li_ref[...] = jnp.broadcast_to(inten, shape)


def kernel(idx, light_direction_xy, light_direction_z, light_intensity):
    params_t = jnp.concatenate(
        [light_direction_xy, light_direction_z, light_intensity], axis=1
    ).T
    idx2 = idx.reshape(_BATCH, 1)
    grid = _BATCH // _NB

    p_ld, p_li = pl.pallas_call(
        _light_kernel,
        grid=(grid,),
        in_specs=[
            pl.BlockSpec((_NB, 1), lambda i: (i, 0)),
            pl.BlockSpec((4, _NUM_LIGHTS), lambda i: (0, 0)),
        ],
        out_specs=[
            pl.BlockSpec((3, _NB, _NUM_RAYS), lambda i: (0, i, 0)),
            pl.BlockSpec((_NB, _NUM_RAYS), lambda i: (i, 0)),
        ],
        out_shape=[
            jax.ShapeDtypeStruct((3, _BATCH, _NUM_RAYS), jnp.float32),
            jax.ShapeDtypeStruct((_BATCH, _NUM_RAYS), jnp.float32),
        ],
    )(idx2, params_t)

    out_ld = p_ld.transpose(1, 2, 0).reshape(-1, 3)
    out_li = jnp.broadcast_to(
        p_li.reshape(_BATCH * _NUM_RAYS, 1), (_BATCH * _NUM_RAYS, 3)
    )
    return (out_ld, out_li)
